# baseline (speedup 1.0000x reference)
import functools

import jax
import jax.numpy as jnp
from jax import lax
from jax.experimental import pallas as pl
from jax.experimental.pallas import tpu as pltpu
from jax.experimental.pallas import tpu_sc as plsc

_LANES = 16
_TILE = 128


def kernel(pos, idx):
    batch, atm, dim = pos.shape
    n_idx = idx.shape[0]
    nrow = dim * batch
    pos2 = pos.transpose(2, 0, 1).reshape(nrow, atm)
    idx32 = idx.astype(jnp.int32)

    info = plsc.get_sparse_core_info()
    nc, ns = info.num_cores, info.num_subcores
    nw = nc * ns
    jpw = n_idx // nw
    opw = jpw * nrow
    steps = nrow // _LANES

    mesh = plsc.VectorSubcoreMesh(core_axis_name="c", subcore_axis_name="s")

    @functools.partial(
        pl.kernel,
        mesh=mesh,
        out_type=jax.ShapeDtypeStruct((n_idx * nrow,), jnp.float32),
        compiler_params=pltpu.CompilerParams(needs_layout_passes=False),
        scratch_types=[
            pltpu.VMEM((n_idx,), jnp.int32),
            [pltpu.VMEM((nrow, _TILE), jnp.float32)] * jpw,
            pltpu.VMEM((opw,), jnp.float32),
            [pltpu.SemaphoreType.DMA] * jpw,
        ],
    )
    def gather_cols(pos_hbm, idx_hbm, out_hbm, idx_v, wins, out_v, sems):
        wid = lax.axis_index("s") * nc + lax.axis_index("c")
        pltpu.sync_copy(idx_hbm, idx_v)
        riota = lax.broadcasted_iota(jnp.int32, (_LANES,), 0)
        lanes = []
        copies = []
        for p in range(jpw):
            j = wid * jpw + p
            chunk = pl.multiple_of((j // _LANES) * _LANES, _LANES)
            ivec = idx_v[pl.ds(chunk, _LANES)]
            a = jnp.max(jnp.where(riota == j % _LANES, ivec, 0))
            c0 = pl.multiple_of((a // _TILE) * _TILE, _TILE)
            lanes.append(a % _TILE)
            copies.append(pltpu.async_copy(
                pos_hbm.at[:, pl.ds(c0, _TILE)], wins[p], sems[p]))
        for p in range(jpw):
            copies[p].wait()
            cvec = jnp.full((_LANES,), lanes[p], jnp.int32)
            for t in range(steps):
                out_v[pl.ds(p * nrow + t * _LANES, _LANES)] = plsc.load_gather(
                    wins[p], [riota + t * _LANES, cvec])
        pltpu.sync_copy(out_v, out_hbm.at[pl.ds(wid * opw, opw)])

    out = gather_cols(pos2, idx32)
    return (out.reshape(n_idx, dim, batch)
            .transpose(2, 0, 1)
            .reshape(batch, n_idx * dim))
